# baseline (speedup 1.0000x reference)
import functools

import jax
import jax.numpy as jnp
from jax import lax
from jax.experimental import pallas as pl
from jax.experimental.pallas import tpu as pltpu
from jax.experimental.pallas import tpu_sc as plsc

F = 26
V = 100000
D = 64
A = 128
B = 4096
FD = F * D
W2 = 2 * D
TEMP = 5.0
REG_WEIGHT = 0.1

_info = plsc.get_sparse_core_info()
_NC, _NS = _info.num_cores, _info.num_subcores
_NW = _NC * _NS
_BW = B // _NW

_CS = (24960, 24960, 24960, 25120)
_COFF = (0, 24960, 49920, 74880)
_CMAX = 25120
_NVC = 4


def _eye():
    return (lax.broadcasted_iota(jnp.int32, (D, D), 0)
            == lax.broadcasted_iota(jnp.int32, (D, D), 1)).astype(jnp.float32)


def _repack_body(t_hbm, out_hbm, b0, b1, o0, o1, is0, is1, os0, os1):
    f = pl.program_id(0)
    c = pl.program_id(1)
    bufs, obufs = (b0, b1), (o0, o1)
    isems, osems = (is0, is1), (os0, os1)
    eye = _eye()

    def in_desc(ff, cc):
        return pltpu.make_async_copy(
            t_hbm.at[ff, :, pl.ds(_COFF[cc], _CS[cc])],
            bufs[cc % 2].at[:, 0 : _CS[cc]],
            isems[cc % 2],
        )

    def out_desc(ff, cc):
        rows = _CS[cc]
        row0 = ff * V + _COFF[cc]
        return pltpu.make_async_copy(
            obufs[cc % 2].at[0:rows, :],
            out_hbm.at[pl.ds(row0, rows), :],
            osems[cc % 2],
        )

    @pl.when(jnp.logical_and(f == 0, c == 0))
    def _():
        in_desc(0, 0).start()

    for cc in range(_NVC):
        @pl.when(c == cc)
        def _(cc=cc):
            in_desc(f, cc).wait()
            if cc < _NVC - 1:
                in_desc(f, cc + 1).start()
            else:
                @pl.when(f < F - 1)
                def _():
                    in_desc(f + 1, 0).start()
            if cc >= 2:
                out_desc(f, cc - 2).wait()
            else:
                @pl.when(f > 0)
                def _(cc=cc):
                    out_desc(f - 1, cc + 2).wait()
            blk = bufs[cc % 2][:, 0 : _CS[cc]]
            t = lax.dot_general(blk, eye, (((0,), (0,)), ((), ())),
                                preferred_element_type=jnp.float32)
            obufs[cc % 2][0 : _CS[cc], 0:D] = t
            out_desc(f, cc).start()

    @pl.when(jnp.logical_and(f == F - 1, c == _NVC - 1))
    def _():
        out_desc(F - 1, _NVC - 2).wait()
        out_desc(F - 1, _NVC - 1).wait()


def _tc_repack(tab_t):
    return pl.pallas_call(
        _repack_body,
        grid=(F, _NVC),
        in_specs=[pl.BlockSpec(memory_space=pl.ANY)],
        out_specs=pl.BlockSpec(memory_space=pl.ANY),
        out_shape=jax.ShapeDtypeStruct((F * V, W2), jnp.float32),
        scratch_shapes=[
            pltpu.VMEM((D, _CMAX), jnp.float32),
            pltpu.VMEM((D, _CMAX), jnp.float32),
            pltpu.VMEM((_CMAX, W2), jnp.float32),
            pltpu.VMEM((_CMAX, W2), jnp.float32),
            pltpu.SemaphoreType.DMA,
            pltpu.SemaphoreType.DMA,
            pltpu.SemaphoreType.DMA,
            pltpu.SemaphoreType.DMA,
        ],
    )(tab_t)


def _sc_gather(tables_rows, idx_flat):
    mesh = plsc.VectorSubcoreMesh(core_axis_name="c", subcore_axis_name="s")

    @functools.partial(
        pl.kernel,
        mesh=mesh,
        out_type=jax.ShapeDtypeStruct((B, F * W2), jnp.float32),
        scratch_types=[
            pltpu.VMEM((F, _BW), jnp.int32),
            pltpu.VMEM((_BW, W2), jnp.float32),
            pltpu.VMEM((_BW, W2), jnp.float32),
            pltpu.SemaphoreType.DMA,
            pltpu.SemaphoreType.DMA,
        ],
    )
    def k(tab_hbm, idx_hbm, x_hbm, idx_v, rows_a, rows_b, sem_a, sem_b):
        wid = lax.axis_index("s") * _NC + lax.axis_index("c")
        base = wid * _BW
        pltpu.sync_copy(idx_hbm.at[:, pl.ds(base, _BW)], idx_v)

        def gat(f, buf, sem):
            return pltpu.make_async_copy(tab_hbm.at[idx_v.at[f]], buf, sem)

        def st(f, buf):
            col = pl.multiple_of(f * W2, W2)
            pltpu.sync_copy(buf, x_hbm.at[pl.ds(base, _BW), pl.ds(col, W2)])

        gat(0, rows_a, sem_a).start()

        def body(j, carry):
            f0 = 2 * j
            gat(f0, rows_a, sem_a).wait()
            gat(f0 + 1, rows_b, sem_b).start()
            st(f0, rows_a)
            gat(f0 + 1, rows_b, sem_b).wait()

            @pl.when(j < F // 2 - 1)
            def _():
                gat(f0 + 2, rows_a, sem_a).start()

            st(f0 + 1, rows_b)
            return carry

        lax.fori_loop(0, F // 2, body, 0)

    return k(tables_rows, idx_flat)


_BT = 512


def _tc_body(x_ref, w2_ref, thw_ref, th_ref, out_ref, fs_ref):
    g2 = jax.nn.sigmoid(thw_ref[...] * TEMP)
    lane = lax.broadcasted_iota(jnp.int32, (_BT, F * W2), 1)
    left = (lane % W2) < D
    xm = jnp.where(left, x_ref[...], 0.0) * g2
    out_ref[...] = jnp.dot(xm, w2_ref[...], preferred_element_type=jnp.float32)

    @pl.when(pl.program_id(0) == 0)
    def _():
        g = jax.nn.sigmoid(th_ref[...] * TEMP)
        m = jnp.mean(g)
        fs = jnp.mean(g - jnp.abs(g - m)) * REG_WEIGHT
        fs_ref[...] = jnp.full((1, 1), fs, jnp.float32)


def _tc_matmul(x_wide, w2, theta_w, theta2):
    return pl.pallas_call(
        _tc_body,
        grid=(B // _BT,),
        in_specs=[
            pl.BlockSpec((_BT, F * W2), lambda i: (i, 0)),
            pl.BlockSpec((F * W2, A), lambda i: (0, 0)),
            pl.BlockSpec((1, F * W2), lambda i: (0, 0)),
            pl.BlockSpec((F, D), lambda i: (0, 0)),
        ],
        out_specs=[
            pl.BlockSpec((_BT, A), lambda i: (i, 0)),
            pl.BlockSpec((1, 1), lambda i: (0, 0)),
        ],
        out_shape=[
            jax.ShapeDtypeStruct((B, A), jnp.float32),
            jax.ShapeDtypeStruct((1, 1), jnp.float32),
        ],
    )(x_wide, w2, theta_w, theta2)


def kernel(inputs, tables, theta, weight):
    tab_t = jnp.transpose(tables, (0, 2, 1))
    tables_rows = _tc_repack(tab_t)
    idx_flat = inputs.T + (jnp.arange(F, dtype=jnp.int32) * V)[:, None]
    theta_w = jnp.pad(
        theta.reshape(F, 1, D), ((0, 0), (0, 0), (0, D)), constant_values=-1e4
    ).reshape(1, F * W2)
    w3 = weight.reshape(F, D, A)
    w2 = jnp.concatenate([w3, jnp.zeros_like(w3)], axis=1).reshape(F * W2, A)
    x_wide = _sc_gather(tables_rows, idx_flat)
    out, fs = _tc_matmul(x_wide, w2, theta_w, theta.reshape(F, D))
    return (out, fs.reshape(()))


# speedup vs baseline: 1.0019x; 1.0019x over previous
import functools

import jax
import jax.numpy as jnp
from jax import lax
from jax.experimental import pallas as pl
from jax.experimental.pallas import tpu as pltpu
from jax.experimental.pallas import tpu_sc as plsc

F = 26
V = 100000
D = 64
A = 128
B = 4096
FD = F * D
W2 = 2 * D
TEMP = 5.0
REG_WEIGHT = 0.1

_info = plsc.get_sparse_core_info()
_NC, _NS = _info.num_cores, _info.num_subcores
_NW = _NC * _NS
_BW = B // _NW

_CS = (24960, 24960, 24960, 25120)
_COFF = (0, 24960, 49920, 74880)
_CMAX = 25120
_NVC = 4


def _eye():
    return (lax.broadcasted_iota(jnp.int32, (D, D), 0)
            == lax.broadcasted_iota(jnp.int32, (D, D), 1)).astype(jnp.float32)


def _repack_body(f0, nf, t_hbm, out_hbm, b0, b1, o0, o1, is0, is1, os0, os1):
    f = pl.program_id(0)
    c = pl.program_id(1)
    bufs, obufs = (b0, b1), (o0, o1)
    isems, osems = (is0, is1), (os0, os1)
    eye = _eye()

    def in_desc(ff, cc):
        return pltpu.make_async_copy(
            t_hbm.at[f0 + ff, :, pl.ds(_COFF[cc], _CS[cc])],
            bufs[cc % 2].at[:, 0 : _CS[cc]],
            isems[cc % 2],
        )

    def out_desc(ff, cc):
        rows = _CS[cc]
        row0 = ff * V + _COFF[cc]
        return pltpu.make_async_copy(
            obufs[cc % 2].at[0:rows, :],
            out_hbm.at[pl.ds(row0, rows), :],
            osems[cc % 2],
        )

    @pl.when(jnp.logical_and(f == 0, c == 0))
    def _():
        in_desc(0, 0).start()

    for cc in range(_NVC):
        @pl.when(c == cc)
        def _(cc=cc):
            in_desc(f, cc).wait()
            if cc < _NVC - 1:
                in_desc(f, cc + 1).start()
            else:
                @pl.when(f < nf - 1)
                def _():
                    in_desc(f + 1, 0).start()
            if cc >= 2:
                out_desc(f, cc - 2).wait()
            else:
                @pl.when(f > 0)
                def _(cc=cc):
                    out_desc(f - 1, cc + 2).wait()
            blk = bufs[cc % 2][:, 0 : _CS[cc]]
            t = lax.dot_general(blk, eye, (((0,), (0,)), ((), ())),
                                preferred_element_type=jnp.float32)
            obufs[cc % 2][0 : _CS[cc], 0:D] = t
            out_desc(f, cc).start()

    @pl.when(jnp.logical_and(f == nf - 1, c == _NVC - 1))
    def _():
        out_desc(nf - 1, _NVC - 2).wait()
        out_desc(nf - 1, _NVC - 1).wait()


def _tc_repack(tab_t, f0, nf):
    return pl.pallas_call(
        functools.partial(_repack_body, f0, nf),
        grid=(nf, _NVC),
        in_specs=[pl.BlockSpec(memory_space=pl.ANY)],
        out_specs=pl.BlockSpec(memory_space=pl.ANY),
        out_shape=jax.ShapeDtypeStruct((nf * V, W2), jnp.float32),
        scratch_shapes=[
            pltpu.VMEM((D, _CMAX), jnp.float32),
            pltpu.VMEM((D, _CMAX), jnp.float32),
            pltpu.VMEM((_CMAX, W2), jnp.float32),
            pltpu.VMEM((_CMAX, W2), jnp.float32),
            pltpu.SemaphoreType.DMA,
            pltpu.SemaphoreType.DMA,
            pltpu.SemaphoreType.DMA,
            pltpu.SemaphoreType.DMA,
        ],
    )(tab_t)


def _sc_gather(tables_rows, idx_flat, nf):
    mesh = plsc.VectorSubcoreMesh(core_axis_name="c", subcore_axis_name="s")

    @functools.partial(
        pl.kernel,
        mesh=mesh,
        out_type=jax.ShapeDtypeStruct((B, nf * W2), jnp.float32),
        scratch_types=[
            pltpu.VMEM((nf, _BW), jnp.int32),
            pltpu.VMEM((_BW, W2), jnp.float32),
            pltpu.VMEM((_BW, W2), jnp.float32),
            pltpu.SemaphoreType.DMA,
            pltpu.SemaphoreType.DMA,
        ],
    )
    def k(tab_hbm, idx_hbm, x_hbm, idx_v, rows_a, rows_b, sem_a, sem_b):
        wid = lax.axis_index("s") * _NC + lax.axis_index("c")
        base = wid * _BW
        pltpu.sync_copy(idx_hbm.at[:, pl.ds(base, _BW)], idx_v)

        def gat(f, buf, sem):
            return pltpu.make_async_copy(tab_hbm.at[idx_v.at[f]], buf, sem)

        def st(f, buf):
            col = pl.multiple_of(f * W2, W2)
            pltpu.sync_copy(buf, x_hbm.at[pl.ds(base, _BW), pl.ds(col, W2)])

        gat(0, rows_a, sem_a).start()

        def body(j, carry):
            f0 = 2 * j
            gat(f0, rows_a, sem_a).wait()
            gat(f0 + 1, rows_b, sem_b).start()
            st(f0, rows_a)
            gat(f0 + 1, rows_b, sem_b).wait()

            @pl.when(j < nf // 2 - 1)
            def _():
                gat(f0 + 2, rows_a, sem_a).start()

            st(f0 + 1, rows_b)
            return carry

        lax.fori_loop(0, nf // 2, body, 0)

    return k(tables_rows, idx_flat)


_BT = 512


def _tc_body(x1_ref, x2_ref, w2_ref, thw_ref, th_ref, out_ref, fs_ref):
    g2 = jax.nn.sigmoid(thw_ref[...] * TEMP)
    lane = lax.broadcasted_iota(jnp.int32, (_BT, F * W2), 1)
    left = (lane % W2) < D
    x = jnp.concatenate([x1_ref[...], x2_ref[...]], axis=1)
    xm = jnp.where(left, x, 0.0) * g2
    out_ref[...] = jnp.dot(xm, w2_ref[...], preferred_element_type=jnp.float32)

    @pl.when(pl.program_id(0) == 0)
    def _():
        g = jax.nn.sigmoid(th_ref[...] * TEMP)
        m = jnp.mean(g)
        fs = jnp.mean(g - jnp.abs(g - m)) * REG_WEIGHT
        fs_ref[...] = jnp.full((1, 1), fs, jnp.float32)


_FH = 14


def _tc_matmul(x1, x2, w2, theta_w, theta2):
    return pl.pallas_call(
        _tc_body,
        grid=(B // _BT,),
        in_specs=[
            pl.BlockSpec((_BT, _FH * W2), lambda i: (i, 0)),
            pl.BlockSpec((_BT, (F - _FH) * W2), lambda i: (i, 0)),
            pl.BlockSpec((F * W2, A), lambda i: (0, 0)),
            pl.BlockSpec((1, F * W2), lambda i: (0, 0)),
            pl.BlockSpec((F, D), lambda i: (0, 0)),
        ],
        out_specs=[
            pl.BlockSpec((_BT, A), lambda i: (i, 0)),
            pl.BlockSpec((1, 1), lambda i: (0, 0)),
        ],
        out_shape=[
            jax.ShapeDtypeStruct((B, A), jnp.float32),
            jax.ShapeDtypeStruct((1, 1), jnp.float32),
        ],
    )(x1, x2, w2, theta_w, theta2)


def kernel(inputs, tables, theta, weight):
    tab_t = jnp.transpose(tables, (0, 2, 1))
    idxT = inputs.T
    idx1 = idxT[:_FH] + (jnp.arange(_FH, dtype=jnp.int32) * V)[:, None]
    idx2 = idxT[_FH:] + (jnp.arange(F - _FH, dtype=jnp.int32) * V)[:, None]
    rows1 = _tc_repack(tab_t, 0, _FH)
    rows2 = _tc_repack(tab_t, _FH, F - _FH)
    x1 = _sc_gather(rows1, idx1, _FH)
    x2 = _sc_gather(rows2, idx2, F - _FH)
    theta_w = jnp.pad(
        theta.reshape(F, 1, D), ((0, 0), (0, 0), (0, D)), constant_values=-1e4
    ).reshape(1, F * W2)
    w3 = weight.reshape(F, D, A)
    w2 = jnp.concatenate([w3, jnp.zeros_like(w3)], axis=1).reshape(F * W2, A)
    out, fs = _tc_matmul(x1, x2, w2, theta_w, theta.reshape(F, D))
    return (out, fs.reshape(()))
